# Initial kernel scaffold; baseline (speedup 1.0000x reference)
import jax
import jax.numpy as jnp
from jax.experimental import pallas as pl


def kernel(x, edge_index, W1, W2, W3, gn1_w, gn1_b, gn1_a, gn2_w, gn2_b, gn2_a, gn3_w, gn3_b, gn3_a, r1_phi_w, r1_phi_b, r1_rho_w, r1_rho_b, r2_phi_w, r2_phi_b, r2_rho_w, r2_rho_b, r3_phi_w, r3_phi_b, r3_rho_w, r3_rho_b):
    raise NotImplementedError("write your pallas kernel here")


# speedup vs baseline: 4.9522x; 4.9522x over previous
import functools

import jax
import jax.numpy as jnp
from jax import lax
from jax.experimental import pallas as pl
from jax.experimental.pallas import tpu as pltpu
from jax.experimental.pallas import tpu_sc as plsc

N = 10000
E = 320000
D = 128
R = D // 2

NC = 2
NS = 16
NW = NC * NS

NP = 10240
RPT = NP // NS
NCH = 80
EPT = NCH * 128

BN = 512
NB = NP // BN


@functools.cache
def _sc_deg():
    mesh = plsc.VectorSubcoreMesh(
        core_axis_name="c", subcore_axis_name="s",
        num_cores=NC, num_subcores=NS)

    @functools.partial(
        pl.kernel,
        out_type=jax.ShapeDtypeStruct((NC, NP, D), jnp.float32),
        mesh=mesh,
        scratch_types=[
            pltpu.VMEM((128,), jnp.int32),
            pltpu.VMEM((128,), jnp.int32),
            pltpu.VMEM((128, D), jnp.float32),
            pltpu.VMEM((128, D), jnp.float32),
            pltpu.VMEM_SHARED((NP, D), jnp.float32),
        ],
    )
    def _deg(srcp, dstp, eo, ei, zrow, dg_out,
             idx_s, idx_d, eo_v, ei_v, deg_sh):
        c = lax.axis_index("c")
        s = lax.axis_index("s")
        w = c * NS + s
        r0 = s * RPT
        pltpu.sync_copy(zrow, eo_v)
        for k in range(RPT // 128):
            pltpu.sync_copy(eo_v, deg_sh.at[pl.ds(r0 + k * 128, 128)])
        pltpu.sync_copy(eo, eo_v)
        pltpu.sync_copy(ei, ei_v)
        plsc.subcore_barrier()

        def body(j, carry):
            base = w * EPT + j * 128
            pltpu.sync_copy(srcp.at[pl.ds(base, 128)], idx_s)
            pltpu.sync_copy(dstp.at[pl.ds(base, 128)], idx_d)
            pltpu.sync_copy(eo_v, deg_sh.at[idx_s], add=True)
            pltpu.sync_copy(ei_v, deg_sh.at[idx_d], add=True)
            return carry

        lax.fori_loop(0, NCH, body, 0)
        plsc.subcore_barrier()
        for k in range(RPT // 128):
            rr = r0 + k * 128
            pltpu.sync_copy(deg_sh.at[pl.ds(rr, 128)],
                            dg_out.at[c, pl.ds(rr, 128)])

    return _deg


@functools.cache
def _sc_agg():
    mesh = plsc.VectorSubcoreMesh(
        core_axis_name="c", subcore_axis_name="s",
        num_cores=NC, num_subcores=NS)

    @functools.partial(
        pl.kernel,
        out_type=jax.ShapeDtypeStruct((NC, NP, D), jnp.float32),
        mesh=mesh,
        scratch_types=[
            pltpu.VMEM((128,), jnp.int32),
            pltpu.VMEM((128,), jnp.int32),
            pltpu.VMEM((128, D), jnp.float32),
            pltpu.VMEM_SHARED((NP, D), jnp.float32),
            pltpu.SemaphoreType.DMA,
        ],
    )
    def _agg(hs, srcp, dstp, zrow, p_out, idx_s, idx_d, buf, agg_sh, sem):
        c = lax.axis_index("c")
        s = lax.axis_index("s")
        w = c * NS + s
        r0 = s * RPT
        pltpu.sync_copy(zrow, buf)
        for k in range(RPT // 128):
            pltpu.sync_copy(buf, agg_sh.at[pl.ds(r0 + k * 128, 128)])
        plsc.subcore_barrier()

        def body(j, carry):
            base = w * EPT + j * 128
            pltpu.sync_copy(srcp.at[pl.ds(base, 128)], idx_s)
            pltpu.sync_copy(dstp.at[pl.ds(base, 128)], idx_d)
            pltpu.async_copy(hs.at[idx_s], buf, sem).wait()
            pltpu.sync_copy(buf, agg_sh.at[idx_d], add=True)
            return carry

        lax.fori_loop(0, NCH, body, 0)
        plsc.subcore_barrier()
        for k in range(RPT // 128):
            rr = r0 + k * 128
            pltpu.sync_copy(agg_sh.at[pl.ds(rr, 128)],
                            p_out.at[c, pl.ds(rr, 128)])

    return _agg


def _lrelu(v):
    return jnp.where(v >= 0, v, 0.01 * v)


def _norm_col(dg, col):
    d = dg[0, :, col:col + 1] + dg[1, :, col:col + 1]
    return jnp.where(d > 0, lax.rsqrt(jnp.maximum(d, 1.0)), 0.0)


def _prep_body(xp_ref, dg_ref, hs_ref):
    hs_ref[...] = xp_ref[...] * _norm_col(dg_ref[...], 0)


_prep = pl.pallas_call(
    _prep_body,
    grid=(NB,),
    in_specs=[
        pl.BlockSpec((BN, D), lambda i: (i, 0)),
        pl.BlockSpec((NC, BN, D), lambda i: (0, i, 0)),
    ],
    out_specs=pl.BlockSpec((BN, D), lambda i: (i, 0)),
    out_shape=jax.ShapeDtypeStruct((NP, D), jnp.float32),
)


def _stats_body(p_ref, dg_ref, w_ref, y_ref, s_ref):
    i = pl.program_id(0)
    agg = (p_ref[0] + p_ref[1]) * _norm_col(dg_ref[...], 1)
    y = jnp.dot(agg, w_ref[...], preferred_element_type=jnp.float32,
                precision=lax.Precision.HIGHEST)
    y_ref[...] = y
    s1 = jnp.sum(y, axis=0, keepdims=True)
    s2 = jnp.sum(y * y, axis=0, keepdims=True)
    s = jnp.concatenate([s1, s2], axis=0)

    @pl.when(i == 0)
    def _():
        s_ref[...] = s

    @pl.when(i > 0)
    def _():
        s_ref[...] += s


_stats = pl.pallas_call(
    _stats_body,
    grid=(NB,),
    in_specs=[
        pl.BlockSpec((NC, BN, D), lambda i: (0, i, 0)),
        pl.BlockSpec((NC, BN, D), lambda i: (0, i, 0)),
        pl.BlockSpec((D, D), lambda i: (0, 0)),
    ],
    out_specs=[
        pl.BlockSpec((BN, D), lambda i: (i, 0)),
        pl.BlockSpec((2, D), lambda i: (0, 0)),
    ],
    out_shape=[
        jax.ShapeDtypeStruct((NP, D), jnp.float32),
        jax.ShapeDtypeStruct((2, D), jnp.float32),
    ],
)


def _norm_body(with_next, y_ref, s_ref, dg_ref, gw_ref, gb_ref, ga_ref,
               pw_ref, pb_ref, *out_refs):
    if with_next:
        hs_ref, z_ref = out_refs
    else:
        (z_ref,) = out_refs
    i = pl.program_id(0)
    inv_n = 1.0 / N
    mean = s_ref[0:1] * inv_n
    ey2 = s_ref[1:2] * inv_n
    a = ga_ref[...]
    var = ey2 - (2.0 * a - a * a) * mean * mean
    inv = lax.rsqrt(var + 1e-5)
    h = gw_ref[...] * (y_ref[...] - a * mean) * inv + gb_ref[...]
    h = _lrelu(h)
    rid = lax.broadcasted_iota(jnp.int32, (BN, 1), 0) + i * BN
    mask = rid < N
    h = jnp.where(mask, h, 0.0)
    z = jnp.dot(h, pw_ref[...], preferred_element_type=jnp.float32,
                precision=lax.Precision.HIGHEST) + pb_ref[...]
    z = jnp.where(mask, _lrelu(z), 0.0)
    zs = jnp.sum(z, axis=0, keepdims=True)

    @pl.when(i == 0)
    def _():
        z_ref[...] = zs

    @pl.when(i > 0)
    def _():
        z_ref[...] += zs

    if with_next:
        hs_ref[...] = h * _norm_col(dg_ref[...], 0)


def _make_norm(with_next):
    out_specs = [pl.BlockSpec((1, D), lambda i: (0, 0))]
    out_shape = [jax.ShapeDtypeStruct((1, D), jnp.float32)]
    if with_next:
        out_specs = [pl.BlockSpec((BN, D), lambda i: (i, 0))] + out_specs
        out_shape = [jax.ShapeDtypeStruct((NP, D), jnp.float32)] + out_shape
    return pl.pallas_call(
        functools.partial(_norm_body, with_next),
        grid=(NB,),
        in_specs=[
            pl.BlockSpec((BN, D), lambda i: (i, 0)),
            pl.BlockSpec((2, D), lambda i: (0, 0)),
            pl.BlockSpec((NC, BN, D), lambda i: (0, i, 0)),
            pl.BlockSpec((1, D), lambda i: (0, 0)),
            pl.BlockSpec((1, D), lambda i: (0, 0)),
            pl.BlockSpec((1, D), lambda i: (0, 0)),
            pl.BlockSpec((D, D), lambda i: (0, 0)),
            pl.BlockSpec((1, D), lambda i: (0, 0)),
        ],
        out_specs=out_specs,
        out_shape=out_shape,
    )


_norm_mid = _make_norm(True)
_norm_last = _make_norm(False)


def _final_body(z1, z2, z3, rw1, rb1, rw2, rb2, rw3, rb3, out_ref):
    def ro(z, rw, rb):
        v = jnp.dot(z[...], rw[...], preferred_element_type=jnp.float32,
                    precision=lax.Precision.HIGHEST) + rb[...]
        return _lrelu(v)

    r = jnp.concatenate([ro(z1, rw1, rb1), ro(z2, rw2, rb2), ro(z3, rw3, rb3)],
                        axis=1)
    out_ref[...] = _lrelu(r)


_final = pl.pallas_call(
    _final_body,
    out_shape=jax.ShapeDtypeStruct((1, 3 * R), jnp.float32),
)


def kernel(x, edge_index, W1, W2, W3, gn1_w, gn1_b, gn1_a, gn2_w, gn2_b,
           gn2_a, gn3_w, gn3_b, gn3_a, r1_phi_w, r1_phi_b, r1_rho_w, r1_rho_b,
           r2_phi_w, r2_phi_b, r2_rho_w, r2_rho_b, r3_phi_w, r3_phi_b,
           r3_rho_w, r3_rho_b):
    src = edge_index[0].astype(jnp.int32)
    dst = edge_index[1].astype(jnp.int32)
    pad_n = NW * EPT - E
    pad_idx = N + (jnp.arange(pad_n, dtype=jnp.int32) % (NP - N))
    srcp = jnp.concatenate([src, pad_idx])
    dstp = jnp.concatenate([dst, pad_idx])

    xp = jnp.zeros((NP, D), jnp.float32).at[:N].set(x)
    zrow = jnp.zeros((128, D), jnp.float32)
    eo = jnp.zeros((128, D), jnp.float32).at[:, 0].set(1.0)
    ei = jnp.zeros((128, D), jnp.float32).at[:, 1].set(1.0)

    row = lambda v: v.reshape(1, -1)

    dg = _sc_deg()(srcp, dstp, eo, ei, zrow)
    hs = _prep(xp, dg)

    p1 = _sc_agg()(hs, srcp, dstp, zrow)
    y1, s1 = _stats(p1, dg, W1)
    hs2, z1 = _norm_mid(y1, s1, dg, row(gn1_w), row(gn1_b), row(gn1_a),
                        r1_phi_w, row(r1_phi_b))

    p2 = _sc_agg()(hs2, srcp, dstp, zrow)
    y2, s2 = _stats(p2, dg, W2)
    hs3, z2 = _norm_mid(y2, s2, dg, row(gn2_w), row(gn2_b), row(gn2_a),
                        r2_phi_w, row(r2_phi_b))

    p3 = _sc_agg()(hs3, srcp, dstp, zrow)
    y3, s3 = _stats(p3, dg, W3)
    (z3,) = _norm_last(y3, s3, dg, row(gn3_w), row(gn3_b), row(gn3_a),
                       r3_phi_w, row(r3_phi_b))

    return _final(z1, z2, z3, r1_rho_w, row(r1_rho_b), r2_rho_w, row(r2_rho_b),
                  r3_rho_w, row(r3_rho_b))


# speedup vs baseline: 8.0120x; 1.6179x over previous
import functools

import jax
import jax.numpy as jnp
from jax import lax
from jax.experimental import pallas as pl
from jax.experimental.pallas import tpu as pltpu
from jax.experimental.pallas import tpu_sc as plsc

N = 10000
E = 320000
D = 128
R = D // 2

NC = 2
NS = 16
NW = NC * NS

NP = 10240
RPT = NP // NS
NCH = 80
EPT = NCH * 128

BN = 512
NB = NP // BN


@functools.cache
def _sc_deg():
    mesh = plsc.VectorSubcoreMesh(
        core_axis_name="c", subcore_axis_name="s",
        num_cores=NC, num_subcores=NS)

    @functools.partial(
        pl.kernel,
        out_type=jax.ShapeDtypeStruct((NC, NP, D), jnp.float32),
        mesh=mesh,
        scratch_types=[
            pltpu.VMEM((128,), jnp.int32),
            pltpu.VMEM((128,), jnp.int32),
            pltpu.VMEM((128, D), jnp.float32),
            pltpu.VMEM((128, D), jnp.float32),
            pltpu.VMEM_SHARED((NP, D), jnp.float32),
        ],
    )
    def _deg(srcp, dstp, eo, ei, zrow, dg_out,
             idx_s, idx_d, eo_v, ei_v, deg_sh):
        c = lax.axis_index("c")
        s = lax.axis_index("s")
        w = c * NS + s
        r0 = s * RPT
        pltpu.sync_copy(zrow, eo_v)
        for k in range(RPT // 128):
            pltpu.sync_copy(eo_v, deg_sh.at[pl.ds(r0 + k * 128, 128)])
        pltpu.sync_copy(eo, eo_v)
        pltpu.sync_copy(ei, ei_v)
        plsc.subcore_barrier()

        def body(j, carry):
            base = w * EPT + j * 128
            pltpu.sync_copy(srcp.at[pl.ds(base, 128)], idx_s)
            pltpu.sync_copy(dstp.at[pl.ds(base, 128)], idx_d)
            pltpu.sync_copy(eo_v, deg_sh.at[idx_s], add=True)
            pltpu.sync_copy(ei_v, deg_sh.at[idx_d], add=True)
            return carry

        lax.fori_loop(0, NCH, body, 0)
        plsc.subcore_barrier()
        for k in range(RPT // 128):
            rr = r0 + k * 128
            pltpu.sync_copy(deg_sh.at[pl.ds(rr, 128)],
                            dg_out.at[c, pl.ds(rr, 128)])

    return _deg


@functools.cache
def _sc_agg():
    mesh = plsc.VectorSubcoreMesh(
        core_axis_name="c", subcore_axis_name="s",
        num_cores=NC, num_subcores=NS)

    @functools.partial(
        pl.kernel,
        out_type=jax.ShapeDtypeStruct((NC, NP, D), jnp.float32),
        mesh=mesh,
        scratch_types=[
            [pltpu.VMEM((128,), jnp.int32) for _ in range(4)],
            [pltpu.VMEM((128,), jnp.int32) for _ in range(4)],
            [pltpu.VMEM((128, D), jnp.float32) for _ in range(2)],
            pltpu.VMEM_SHARED((NP, D), jnp.float32),
            pltpu.SemaphoreType.DMA,
            pltpu.SemaphoreType.DMA,
            pltpu.SemaphoreType.DMA,
            pltpu.SemaphoreType.DMA,
        ],
    )
    def _agg(hs, srcp, dstp, zrow, p_out, idx_s, idx_d, buf, agg_sh,
             sem_g, sem_s, sem_i0, sem_i1):
        c = lax.axis_index("c")
        s = lax.axis_index("s")
        w = c * NS + s
        r0 = s * RPT
        e0 = w * EPT
        isems = (sem_i0, sem_i1)

        pltpu.sync_copy(zrow, buf[0])
        for k in range(RPT // 128):
            pltpu.sync_copy(buf[0], agg_sh.at[pl.ds(r0 + k * 128, 128)])
        plsc.subcore_barrier()

        def idx_start(j, slot, par):
            pltpu.async_copy(srcp.at[pl.ds(e0 + j * 128, 128)], idx_s[slot],
                             isems[par])
            pltpu.async_copy(dstp.at[pl.ds(e0 + j * 128, 128)], idx_d[slot],
                             isems[par])

        def idx_wait(slot, par):
            pltpu.make_async_copy(srcp.at[pl.ds(0, 128)], idx_s[slot],
                                  isems[par]).wait()
            pltpu.make_async_copy(dstp.at[pl.ds(0, 128)], idx_d[slot],
                                  isems[par]).wait()

        def g_start(slot, bslot):
            pltpu.async_copy(hs.at[idx_s[slot]], buf[bslot], sem_g)

        def g_wait(slot, bslot):
            pltpu.make_async_copy(hs.at[idx_s[slot]], buf[bslot], sem_g).wait()

        def s_start(slot, bslot):
            pltpu.async_copy(buf[bslot], agg_sh.at[idx_d[slot]], sem_s,
                             add=True)

        def s_wait(slot, bslot):
            pltpu.make_async_copy(buf[bslot], agg_sh.at[idx_d[slot]],
                                  sem_s).wait()

        idx_start(0, 0, 0)
        idx_start(1, 1, 1)
        idx_wait(0, 0)
        g_start(0, 0)

        def body(jj, carry):
            for b in (0, 1, 2, 3):
                j = 4 * jj + b
                cur = b
                nxt = (b + 1) % 4
                pf = (b + 2) % 4
                par = b % 2
                bcur = b % 2
                bnxt = (b + 1) % 2

                if b == 0:
                    @pl.when(j > 0)
                    def _():
                        s_wait(3, bnxt)
                else:
                    s_wait(b - 1, bnxt)

                @pl.when(j + 2 < NCH)
                def _():
                    idx_start(j + 2, pf, par)

                @pl.when(j + 1 < NCH)
                def _():
                    idx_wait(nxt, 1 - par)
                    g_start(nxt, bnxt)

                g_wait(cur, bcur)
                s_start(cur, bcur)
            return carry

        lax.fori_loop(0, NCH // 4, body, 0)
        s_wait((NCH - 1) % 4, (NCH - 1) % 2)
        plsc.subcore_barrier()
        for k in range(RPT // 128):
            rr = r0 + k * 128
            pltpu.sync_copy(agg_sh.at[pl.ds(rr, 128)],
                            p_out.at[c, pl.ds(rr, 128)])

    return _agg


def _lrelu(v):
    return jnp.where(v >= 0, v, 0.01 * v)


def _norm_col(dg, col):
    d = dg[0, :, col:col + 1] + dg[1, :, col:col + 1]
    return jnp.where(d > 0, lax.rsqrt(jnp.maximum(d, 1.0)), 0.0)


def _prep_body(xp_ref, dg_ref, hs_ref):
    hs_ref[...] = xp_ref[...] * _norm_col(dg_ref[...], 0)


_prep = pl.pallas_call(
    _prep_body,
    grid=(NB,),
    in_specs=[
        pl.BlockSpec((BN, D), lambda i: (i, 0)),
        pl.BlockSpec((NC, BN, D), lambda i: (0, i, 0)),
    ],
    out_specs=pl.BlockSpec((BN, D), lambda i: (i, 0)),
    out_shape=jax.ShapeDtypeStruct((NP, D), jnp.float32),
)


def _stats_body(p_ref, dg_ref, w_ref, y_ref, s_ref):
    i = pl.program_id(0)
    agg = (p_ref[0] + p_ref[1]) * _norm_col(dg_ref[...], 1)
    y = jnp.dot(agg, w_ref[...], preferred_element_type=jnp.float32,
                precision=lax.Precision.HIGHEST)
    y_ref[...] = y
    s1 = jnp.sum(y, axis=0, keepdims=True)
    s2 = jnp.sum(y * y, axis=0, keepdims=True)
    s = jnp.concatenate([s1, s2], axis=0)

    @pl.when(i == 0)
    def _():
        s_ref[...] = s

    @pl.when(i > 0)
    def _():
        s_ref[...] += s


_stats = pl.pallas_call(
    _stats_body,
    grid=(NB,),
    in_specs=[
        pl.BlockSpec((NC, BN, D), lambda i: (0, i, 0)),
        pl.BlockSpec((NC, BN, D), lambda i: (0, i, 0)),
        pl.BlockSpec((D, D), lambda i: (0, 0)),
    ],
    out_specs=[
        pl.BlockSpec((BN, D), lambda i: (i, 0)),
        pl.BlockSpec((2, D), lambda i: (0, 0)),
    ],
    out_shape=[
        jax.ShapeDtypeStruct((NP, D), jnp.float32),
        jax.ShapeDtypeStruct((2, D), jnp.float32),
    ],
)


def _norm_body(with_next, y_ref, s_ref, dg_ref, gw_ref, gb_ref, ga_ref,
               pw_ref, pb_ref, *out_refs):
    if with_next:
        hs_ref, z_ref = out_refs
    else:
        (z_ref,) = out_refs
    i = pl.program_id(0)
    inv_n = 1.0 / N
    mean = s_ref[0:1] * inv_n
    ey2 = s_ref[1:2] * inv_n
    a = ga_ref[...]
    var = ey2 - (2.0 * a - a * a) * mean * mean
    inv = lax.rsqrt(var + 1e-5)
    h = gw_ref[...] * (y_ref[...] - a * mean) * inv + gb_ref[...]
    h = _lrelu(h)
    rid = lax.broadcasted_iota(jnp.int32, (BN, 1), 0) + i * BN
    mask = rid < N
    h = jnp.where(mask, h, 0.0)
    z = jnp.dot(h, pw_ref[...], preferred_element_type=jnp.float32,
                precision=lax.Precision.HIGHEST) + pb_ref[...]
    z = jnp.where(mask, _lrelu(z), 0.0)
    zs = jnp.sum(z, axis=0, keepdims=True)

    @pl.when(i == 0)
    def _():
        z_ref[...] = zs

    @pl.when(i > 0)
    def _():
        z_ref[...] += zs

    if with_next:
        hs_ref[...] = h * _norm_col(dg_ref[...], 0)


def _make_norm(with_next):
    out_specs = [pl.BlockSpec((1, D), lambda i: (0, 0))]
    out_shape = [jax.ShapeDtypeStruct((1, D), jnp.float32)]
    if with_next:
        out_specs = [pl.BlockSpec((BN, D), lambda i: (i, 0))] + out_specs
        out_shape = [jax.ShapeDtypeStruct((NP, D), jnp.float32)] + out_shape
    return pl.pallas_call(
        functools.partial(_norm_body, with_next),
        grid=(NB,),
        in_specs=[
            pl.BlockSpec((BN, D), lambda i: (i, 0)),
            pl.BlockSpec((2, D), lambda i: (0, 0)),
            pl.BlockSpec((NC, BN, D), lambda i: (0, i, 0)),
            pl.BlockSpec((1, D), lambda i: (0, 0)),
            pl.BlockSpec((1, D), lambda i: (0, 0)),
            pl.BlockSpec((1, D), lambda i: (0, 0)),
            pl.BlockSpec((D, D), lambda i: (0, 0)),
            pl.BlockSpec((1, D), lambda i: (0, 0)),
        ],
        out_specs=out_specs,
        out_shape=out_shape,
    )


_norm_mid = _make_norm(True)
_norm_last = _make_norm(False)


def _final_body(z1, z2, z3, rw1, rb1, rw2, rb2, rw3, rb3, out_ref):
    def ro(z, rw, rb):
        v = jnp.dot(z[...], rw[...], preferred_element_type=jnp.float32,
                    precision=lax.Precision.HIGHEST) + rb[...]
        return _lrelu(v)

    r = jnp.concatenate([ro(z1, rw1, rb1), ro(z2, rw2, rb2), ro(z3, rw3, rb3)],
                        axis=1)
    out_ref[...] = _lrelu(r)


_final = pl.pallas_call(
    _final_body,
    out_shape=jax.ShapeDtypeStruct((1, 3 * R), jnp.float32),
)


def kernel(x, edge_index, W1, W2, W3, gn1_w, gn1_b, gn1_a, gn2_w, gn2_b,
           gn2_a, gn3_w, gn3_b, gn3_a, r1_phi_w, r1_phi_b, r1_rho_w, r1_rho_b,
           r2_phi_w, r2_phi_b, r2_rho_w, r2_rho_b, r3_phi_w, r3_phi_b,
           r3_rho_w, r3_rho_b):
    src = edge_index[0].astype(jnp.int32)
    dst = edge_index[1].astype(jnp.int32)
    pad_n = NW * EPT - E
    pad_idx = N + (jnp.arange(pad_n, dtype=jnp.int32) % (NP - N))
    srcp = jnp.concatenate([src, pad_idx])
    dstp = jnp.concatenate([dst, pad_idx])

    xp = jnp.zeros((NP, D), jnp.float32).at[:N].set(x)
    zrow = jnp.zeros((128, D), jnp.float32)
    eo = jnp.zeros((128, D), jnp.float32).at[:, 0].set(1.0)
    ei = jnp.zeros((128, D), jnp.float32).at[:, 1].set(1.0)

    row = lambda v: v.reshape(1, -1)

    dg = _sc_deg()(srcp, dstp, eo, ei, zrow)
    hs = _prep(xp, dg)

    p1 = _sc_agg()(hs, srcp, dstp, zrow)
    y1, s1 = _stats(p1, dg, W1)
    hs2, z1 = _norm_mid(y1, s1, dg, row(gn1_w), row(gn1_b), row(gn1_a),
                        r1_phi_w, row(r1_phi_b))

    p2 = _sc_agg()(hs2, srcp, dstp, zrow)
    y2, s2 = _stats(p2, dg, W2)
    hs3, z2 = _norm_mid(y2, s2, dg, row(gn2_w), row(gn2_b), row(gn2_a),
                        r2_phi_w, row(r2_phi_b))

    p3 = _sc_agg()(hs3, srcp, dstp, zrow)
    y3, s3 = _stats(p3, dg, W3)
    (z3,) = _norm_last(y3, s3, dg, row(gn3_w), row(gn3_b), row(gn3_a),
                       r3_phi_w, row(r3_phi_b))

    return _final(z1, z2, z3, r1_rho_w, row(r1_rho_b), r2_rho_w, row(r2_rho_b),
                  r3_rho_w, row(r3_rho_b))


# speedup vs baseline: 8.8958x; 1.1103x over previous
import functools

import jax
import jax.numpy as jnp
from jax import lax
from jax.experimental import pallas as pl
from jax.experimental.pallas import tpu as pltpu
from jax.experimental.pallas import tpu_sc as plsc

N = 10000
E = 320000
D = 128
R = D // 2

NC = 2
NS = 16
NW = NC * NS

NP = 10240
RPT = NP // NS
NCH = 80
EPT = NCH * 128

BN = 512
NB = NP // BN


@functools.cache
def _sc_deg():
    mesh = plsc.VectorSubcoreMesh(
        core_axis_name="c", subcore_axis_name="s",
        num_cores=NC, num_subcores=NS)

    @functools.partial(
        pl.kernel,
        out_type=jax.ShapeDtypeStruct((NC, NP, D), jnp.float32),
        mesh=mesh,
        scratch_types=[
            [pltpu.VMEM((128,), jnp.int32) for _ in range(4)],
            [pltpu.VMEM((128,), jnp.int32) for _ in range(4)],
            pltpu.VMEM((128, D), jnp.float32),
            pltpu.VMEM((128, D), jnp.float32),
            pltpu.VMEM_SHARED((NP, D), jnp.float32),
            pltpu.SemaphoreType.DMA,
            pltpu.SemaphoreType.DMA,
            pltpu.SemaphoreType.DMA,
        ],
    )
    def _deg(srcp, dstp, eo, ei, zrow, dg_out,
             idx_s, idx_d, eo_v, ei_v, deg_sh, sem_s, sem_i0, sem_i1):
        c = lax.axis_index("c")
        s = lax.axis_index("s")
        w = c * NS + s
        r0 = s * RPT
        e0 = w * EPT
        isems = (sem_i0, sem_i1)
        pltpu.sync_copy(zrow, eo_v)
        for k in range(RPT // 128):
            pltpu.sync_copy(eo_v, deg_sh.at[pl.ds(r0 + k * 128, 128)])
        pltpu.sync_copy(eo, eo_v)
        pltpu.sync_copy(ei, ei_v)
        plsc.subcore_barrier()

        def idx_start(j, slot, par):
            pltpu.async_copy(srcp.at[pl.ds(e0 + j * 128, 128)], idx_s[slot],
                             isems[par])
            pltpu.async_copy(dstp.at[pl.ds(e0 + j * 128, 128)], idx_d[slot],
                             isems[par])

        def idx_wait(slot, par):
            pltpu.make_async_copy(srcp.at[pl.ds(0, 128)], idx_s[slot],
                                  isems[par]).wait()
            pltpu.make_async_copy(dstp.at[pl.ds(0, 128)], idx_d[slot],
                                  isems[par]).wait()

        def s_start(slot):
            pltpu.async_copy(eo_v, deg_sh.at[idx_s[slot]], sem_s, add=True)
            pltpu.async_copy(ei_v, deg_sh.at[idx_d[slot]], sem_s, add=True)

        def s_wait(slot):
            pltpu.make_async_copy(eo_v, deg_sh.at[idx_s[slot]], sem_s).wait()
            pltpu.make_async_copy(ei_v, deg_sh.at[idx_d[slot]], sem_s).wait()

        idx_start(0, 0, 0)
        idx_start(1, 1, 1)

        def body(jj, carry):
            for b in (0, 1, 2, 3):
                j = 4 * jj + b
                par = b % 2
                pf = (b + 2) % 4

                if b < 2:
                    @pl.when(j > 1)
                    def _():
                        s_wait(pf)
                else:
                    s_wait(pf)

                @pl.when(j + 2 < NCH)
                def _():
                    idx_start(j + 2, pf, par)

                idx_wait(b, par)
                s_start(b)
            return carry

        lax.fori_loop(0, NCH // 4, body, 0)
        s_wait((NCH - 2) % 4)
        s_wait((NCH - 1) % 4)
        plsc.subcore_barrier()
        for k in range(RPT // 128):
            rr = r0 + k * 128
            pltpu.sync_copy(deg_sh.at[pl.ds(rr, 128)],
                            dg_out.at[c, pl.ds(rr, 128)])

    return _deg


@functools.cache
def _sc_agg():
    mesh = plsc.VectorSubcoreMesh(
        core_axis_name="c", subcore_axis_name="s",
        num_cores=NC, num_subcores=NS)

    @functools.partial(
        pl.kernel,
        out_type=jax.ShapeDtypeStruct((NC, NP, D), jnp.float32),
        mesh=mesh,
        scratch_types=[
            [pltpu.VMEM((128,), jnp.int32) for _ in range(4)],
            [pltpu.VMEM((128,), jnp.int32) for _ in range(4)],
            [pltpu.VMEM((128, D), jnp.float32) for _ in range(2)],
            pltpu.VMEM_SHARED((NP, D), jnp.float32),
            pltpu.SemaphoreType.DMA,
            pltpu.SemaphoreType.DMA,
            pltpu.SemaphoreType.DMA,
            pltpu.SemaphoreType.DMA,
        ],
    )
    def _agg(hs, srcp, dstp, zrow, p_out, idx_s, idx_d, buf, agg_sh,
             sem_g, sem_s, sem_i0, sem_i1):
        c = lax.axis_index("c")
        s = lax.axis_index("s")
        w = c * NS + s
        r0 = s * RPT
        e0 = w * EPT
        isems = (sem_i0, sem_i1)

        pltpu.sync_copy(zrow, buf[0])
        for k in range(RPT // 128):
            pltpu.sync_copy(buf[0], agg_sh.at[pl.ds(r0 + k * 128, 128)])
        plsc.subcore_barrier()

        def idx_start(j, slot, par):
            pltpu.async_copy(srcp.at[pl.ds(e0 + j * 128, 128)], idx_s[slot],
                             isems[par])
            pltpu.async_copy(dstp.at[pl.ds(e0 + j * 128, 128)], idx_d[slot],
                             isems[par])

        def idx_wait(slot, par):
            pltpu.make_async_copy(srcp.at[pl.ds(0, 128)], idx_s[slot],
                                  isems[par]).wait()
            pltpu.make_async_copy(dstp.at[pl.ds(0, 128)], idx_d[slot],
                                  isems[par]).wait()

        def g_start(slot, bslot):
            pltpu.async_copy(hs.at[idx_s[slot]], buf[bslot], sem_g)

        def g_wait(slot, bslot):
            pltpu.make_async_copy(hs.at[idx_s[slot]], buf[bslot], sem_g).wait()

        def s_start(slot, bslot):
            pltpu.async_copy(buf[bslot], agg_sh.at[idx_d[slot]], sem_s,
                             add=True)

        def s_wait(slot, bslot):
            pltpu.make_async_copy(buf[bslot], agg_sh.at[idx_d[slot]],
                                  sem_s).wait()

        idx_start(0, 0, 0)
        idx_start(1, 1, 1)
        idx_wait(0, 0)
        g_start(0, 0)

        def body(jj, carry):
            for b in (0, 1, 2, 3):
                j = 4 * jj + b
                cur = b
                nxt = (b + 1) % 4
                pf = (b + 2) % 4
                par = b % 2
                bcur = b % 2
                bnxt = (b + 1) % 2

                if b == 0:
                    @pl.when(j > 0)
                    def _():
                        s_wait(3, bnxt)
                else:
                    s_wait(b - 1, bnxt)

                @pl.when(j + 2 < NCH)
                def _():
                    idx_start(j + 2, pf, par)

                @pl.when(j + 1 < NCH)
                def _():
                    idx_wait(nxt, 1 - par)
                    g_start(nxt, bnxt)

                g_wait(cur, bcur)
                s_start(cur, bcur)
            return carry

        lax.fori_loop(0, NCH // 4, body, 0)
        s_wait((NCH - 1) % 4, (NCH - 1) % 2)
        plsc.subcore_barrier()
        for k in range(RPT // 128):
            rr = r0 + k * 128
            pltpu.sync_copy(agg_sh.at[pl.ds(rr, 128)],
                            p_out.at[c, pl.ds(rr, 128)])

    return _agg


def _lrelu(v):
    return jnp.where(v >= 0, v, 0.01 * v)


def _norm_col(dg, col):
    d = dg[0, :, col:col + 1] + dg[1, :, col:col + 1]
    return jnp.where(d > 0, lax.rsqrt(jnp.maximum(d, 1.0)), 0.0)


def _prep_body(xp_ref, dg_ref, hs_ref):
    hs_ref[...] = xp_ref[...] * _norm_col(dg_ref[...], 0)


_prep = pl.pallas_call(
    _prep_body,
    grid=(NB,),
    in_specs=[
        pl.BlockSpec((BN, D), lambda i: (i, 0)),
        pl.BlockSpec((NC, BN, D), lambda i: (0, i, 0)),
    ],
    out_specs=pl.BlockSpec((BN, D), lambda i: (i, 0)),
    out_shape=jax.ShapeDtypeStruct((NP, D), jnp.float32),
)


def _stats_body(p_ref, dg_ref, w_ref, y_ref, s_ref):
    i = pl.program_id(0)
    agg = (p_ref[0] + p_ref[1]) * _norm_col(dg_ref[...], 1)
    y = jnp.dot(agg, w_ref[...], preferred_element_type=jnp.float32,
                precision=lax.Precision.HIGHEST)
    y_ref[...] = y
    s1 = jnp.sum(y, axis=0, keepdims=True)
    s2 = jnp.sum(y * y, axis=0, keepdims=True)
    s = jnp.concatenate([s1, s2], axis=0)

    @pl.when(i == 0)
    def _():
        s_ref[...] = s

    @pl.when(i > 0)
    def _():
        s_ref[...] += s


_stats = pl.pallas_call(
    _stats_body,
    grid=(NB,),
    in_specs=[
        pl.BlockSpec((NC, BN, D), lambda i: (0, i, 0)),
        pl.BlockSpec((NC, BN, D), lambda i: (0, i, 0)),
        pl.BlockSpec((D, D), lambda i: (0, 0)),
    ],
    out_specs=[
        pl.BlockSpec((BN, D), lambda i: (i, 0)),
        pl.BlockSpec((2, D), lambda i: (0, 0)),
    ],
    out_shape=[
        jax.ShapeDtypeStruct((NP, D), jnp.float32),
        jax.ShapeDtypeStruct((2, D), jnp.float32),
    ],
)


def _norm_body(with_next, y_ref, s_ref, dg_ref, gw_ref, gb_ref, ga_ref,
               pw_ref, pb_ref, *out_refs):
    if with_next:
        hs_ref, z_ref = out_refs
    else:
        (z_ref,) = out_refs
    i = pl.program_id(0)
    inv_n = 1.0 / N
    mean = s_ref[0:1] * inv_n
    ey2 = s_ref[1:2] * inv_n
    a = ga_ref[...]
    var = ey2 - (2.0 * a - a * a) * mean * mean
    inv = lax.rsqrt(var + 1e-5)
    h = gw_ref[...] * (y_ref[...] - a * mean) * inv + gb_ref[...]
    h = _lrelu(h)
    rid = lax.broadcasted_iota(jnp.int32, (BN, 1), 0) + i * BN
    mask = rid < N
    h = jnp.where(mask, h, 0.0)
    z = jnp.dot(h, pw_ref[...], preferred_element_type=jnp.float32,
                precision=lax.Precision.HIGHEST) + pb_ref[...]
    z = jnp.where(mask, _lrelu(z), 0.0)
    zs = jnp.sum(z, axis=0, keepdims=True)

    @pl.when(i == 0)
    def _():
        z_ref[...] = zs

    @pl.when(i > 0)
    def _():
        z_ref[...] += zs

    if with_next:
        hs_ref[...] = h * _norm_col(dg_ref[...], 0)


def _make_norm(with_next):
    out_specs = [pl.BlockSpec((1, D), lambda i: (0, 0))]
    out_shape = [jax.ShapeDtypeStruct((1, D), jnp.float32)]
    if with_next:
        out_specs = [pl.BlockSpec((BN, D), lambda i: (i, 0))] + out_specs
        out_shape = [jax.ShapeDtypeStruct((NP, D), jnp.float32)] + out_shape
    return pl.pallas_call(
        functools.partial(_norm_body, with_next),
        grid=(NB,),
        in_specs=[
            pl.BlockSpec((BN, D), lambda i: (i, 0)),
            pl.BlockSpec((2, D), lambda i: (0, 0)),
            pl.BlockSpec((NC, BN, D), lambda i: (0, i, 0)),
            pl.BlockSpec((1, D), lambda i: (0, 0)),
            pl.BlockSpec((1, D), lambda i: (0, 0)),
            pl.BlockSpec((1, D), lambda i: (0, 0)),
            pl.BlockSpec((D, D), lambda i: (0, 0)),
            pl.BlockSpec((1, D), lambda i: (0, 0)),
        ],
        out_specs=out_specs,
        out_shape=out_shape,
    )


_norm_mid = _make_norm(True)
_norm_last = _make_norm(False)


def _final_body(z1, z2, z3, rw1, rb1, rw2, rb2, rw3, rb3, out_ref):
    def ro(z, rw, rb):
        v = jnp.dot(z[...], rw[...], preferred_element_type=jnp.float32,
                    precision=lax.Precision.HIGHEST) + rb[...]
        return _lrelu(v)

    r = jnp.concatenate([ro(z1, rw1, rb1), ro(z2, rw2, rb2), ro(z3, rw3, rb3)],
                        axis=1)
    out_ref[...] = _lrelu(r)


_final = pl.pallas_call(
    _final_body,
    out_shape=jax.ShapeDtypeStruct((1, 3 * R), jnp.float32),
)


def kernel(x, edge_index, W1, W2, W3, gn1_w, gn1_b, gn1_a, gn2_w, gn2_b,
           gn2_a, gn3_w, gn3_b, gn3_a, r1_phi_w, r1_phi_b, r1_rho_w, r1_rho_b,
           r2_phi_w, r2_phi_b, r2_rho_w, r2_rho_b, r3_phi_w, r3_phi_b,
           r3_rho_w, r3_rho_b):
    src = edge_index[0].astype(jnp.int32)
    dst = edge_index[1].astype(jnp.int32)
    pad_n = NW * EPT - E
    pad_idx = N + (jnp.arange(pad_n, dtype=jnp.int32) % (NP - N))
    srcp = jnp.concatenate([src, pad_idx])
    dstp = jnp.concatenate([dst, pad_idx])

    xp = jnp.zeros((NP, D), jnp.float32).at[:N].set(x)
    zrow = jnp.zeros((128, D), jnp.float32)
    eo = jnp.zeros((128, D), jnp.float32).at[:, 0].set(1.0)
    ei = jnp.zeros((128, D), jnp.float32).at[:, 1].set(1.0)

    row = lambda v: v.reshape(1, -1)

    dg = _sc_deg()(srcp, dstp, eo, ei, zrow)
    hs = _prep(xp, dg)

    p1 = _sc_agg()(hs, srcp, dstp, zrow)
    y1, s1 = _stats(p1, dg, W1)
    hs2, z1 = _norm_mid(y1, s1, dg, row(gn1_w), row(gn1_b), row(gn1_a),
                        r1_phi_w, row(r1_phi_b))

    p2 = _sc_agg()(hs2, srcp, dstp, zrow)
    y2, s2 = _stats(p2, dg, W2)
    hs3, z2 = _norm_mid(y2, s2, dg, row(gn2_w), row(gn2_b), row(gn2_a),
                        r2_phi_w, row(r2_phi_b))

    p3 = _sc_agg()(hs3, srcp, dstp, zrow)
    y3, s3 = _stats(p3, dg, W3)
    (z3,) = _norm_last(y3, s3, dg, row(gn3_w), row(gn3_b), row(gn3_a),
                       r3_phi_w, row(r3_phi_b))

    return _final(z1, z2, z3, r1_rho_w, row(r1_rho_b), r2_rho_w, row(r2_rho_b),
                  r3_rho_w, row(r3_rho_b))
